# Initial kernel scaffold; baseline (speedup 1.0000x reference)
import jax
import jax.numpy as jnp
from jax.experimental import pallas as pl


def kernel(sequence, segment_labels, token_emb, position_emb, segment_emb):
    raise NotImplementedError("write your pallas kernel here")


# speedup vs baseline: 2.2470x; 2.2470x over previous
import functools

import jax
import jax.numpy as jnp
from jax import lax
from jax.experimental import pallas as pl
from jax.experimental.pallas import tpu as pltpu
from jax.experimental.pallas import tpu_sc as plsc

VOCAB = 1000000
EMBED = 64
SEQ = 200
BATCH = 4096

NC, NS = 2, 16
NW = NC * NS
N = BATCH * SEQ
CHUNK = 128
CHUNKS_PER_W = N // (NW * CHUNK)


def _sc_embed(seq2d, lab2d, token_emb, ps_table):
  mesh = plsc.VectorSubcoreMesh(core_axis_name="c", subcore_axis_name="s")

  @functools.partial(
      pl.kernel,
      mesh=mesh,
      compiler_params=pltpu.CompilerParams(use_tc_tiling_on_sc=False),
      out_type=jax.ShapeDtypeStruct((N, EMBED), jnp.float32),
      scratch_types=[
          pltpu.VMEM((CHUNKS_PER_W, CHUNK), jnp.int32),
          pltpu.VMEM((CHUNKS_PER_W, CHUNK), jnp.int32),
          pltpu.VMEM((CHUNK, EMBED), jnp.float32),
          pltpu.VMEM((CHUNK, EMBED), jnp.float32),
          pltpu.SemaphoreType.DMA,
          pltpu.SemaphoreType.DMA,
      ],
  )
  def k(seq_hbm, lab_hbm, tok_hbm, ps_hbm, out_hbm,
        idx_v, psi_v, tok_rows, ps_rows, sem0, sem1):
    wid = lax.axis_index("s") * NC + lax.axis_index("c")
    row0 = wid * CHUNKS_PER_W
    flat0 = row0 * CHUNK

    pltpu.sync_copy(seq_hbm.at[pl.ds(row0, CHUNKS_PER_W)], idx_v)
    pltpu.sync_copy(lab_hbm.at[pl.ds(row0, CHUNKS_PER_W)], psi_v)

    def fuse(j, carry):
      base = flat0 + j * CHUNK
      for i in range(CHUNK // 16):
        sl = pl.ds(i * 16, 16)
        lanes = base + i * 16 + lax.iota(jnp.int32, 16)
        psi_v[j, sl] = (lanes % SEQ) * 3 + psi_v[j, sl]
      return carry
    lax.fori_loop(0, CHUNKS_PER_W, fuse, 0)

    def chunk(j, carry):
      ct = pltpu.async_copy(tok_hbm.at[idx_v.at[j]], tok_rows, sem0)
      cp = pltpu.async_copy(ps_hbm.at[psi_v.at[j]], ps_rows, sem1)
      ct.wait()
      cp.wait()

      def addr(r, c2):
        for c in range(EMBED // 16):
          sl = pl.ds(c * 16, 16)
          tok_rows[r, sl] = tok_rows[r, sl] + ps_rows[r, sl]
        return c2
      lax.fori_loop(0, CHUNK, addr, 0)

      pltpu.sync_copy(tok_rows, out_hbm.at[pl.ds(flat0 + j * CHUNK, CHUNK)])
      return carry
    lax.fori_loop(0, CHUNKS_PER_W, chunk, 0)

  return k(seq2d, lab2d, token_emb, ps_table)


def kernel(sequence, segment_labels, token_emb, position_emb, segment_emb):
  ps_table = (position_emb[:, None, :] + segment_emb[None, :, :]).reshape(
      SEQ * 3, EMBED)
  seq2d = sequence.reshape(N // CHUNK, CHUNK)
  lab2d = segment_labels.reshape(N // CHUNK, CHUNK)
  out = _sc_embed(seq2d, lab2d, token_emb, ps_table)
  return out.reshape(BATCH, SEQ, EMBED)


# speedup vs baseline: 2.3264x; 1.0353x over previous
import functools

import jax
import jax.numpy as jnp
from jax import lax
from jax.experimental import pallas as pl
from jax.experimental.pallas import tpu as pltpu
from jax.experimental.pallas import tpu_sc as plsc

VOCAB = 1000000
EMBED = 64
SEQ = 200
BATCH = 4096

NC, NS = 2, 16
NW = NC * NS
N = BATCH * SEQ
CHUNK = 128
CHUNKS_PER_W = N // (NW * CHUNK)


def _sc_embed(seq2d, lab2d, token_emb, ps_table):
  mesh = plsc.VectorSubcoreMesh(core_axis_name="c", subcore_axis_name="s")

  @functools.partial(
      pl.kernel,
      mesh=mesh,
      compiler_params=pltpu.CompilerParams(use_tc_tiling_on_sc=False),
      out_type=jax.ShapeDtypeStruct((N, EMBED), jnp.float32),
      scratch_types=[
          pltpu.VMEM((CHUNKS_PER_W, CHUNK), jnp.int32),
          pltpu.VMEM((CHUNKS_PER_W, CHUNK), jnp.int32),
          pltpu.VMEM((CHUNK, EMBED), jnp.float32),
          pltpu.VMEM((CHUNK, EMBED), jnp.float32),
          pltpu.VMEM((CHUNK, EMBED), jnp.float32),
          pltpu.VMEM((CHUNK, EMBED), jnp.float32),
          pltpu.SemaphoreType.DMA,
          pltpu.SemaphoreType.DMA,
      ],
  )
  def k(seq_hbm, lab_hbm, tok_hbm, ps_hbm, out_hbm,
        idx_v, psi_v, tok0, ps0, tok1, ps1, sem0, sem1):
    wid = lax.axis_index("s") * NC + lax.axis_index("c")
    row0 = wid * CHUNKS_PER_W
    flat0 = row0 * CHUNK

    pltpu.sync_copy(seq_hbm.at[pl.ds(row0, CHUNKS_PER_W)], idx_v)
    pltpu.sync_copy(lab_hbm.at[pl.ds(row0, CHUNKS_PER_W)], psi_v)

    def fuse(j, carry):
      base = flat0 + j * CHUNK
      for i in range(CHUNK // 16):
        sl = pl.ds(i * 16, 16)
        lanes = base + i * 16 + lax.iota(jnp.int32, 16)
        psi_v[j, sl] = (lanes % SEQ) * 3 + psi_v[j, sl]
      return carry
    lax.fori_loop(0, CHUNKS_PER_W, fuse, 0)

    def start(j, tok_b, ps_b, sem):
      pltpu.async_copy(tok_hbm.at[idx_v.at[j]], tok_b, sem)
      pltpu.async_copy(ps_hbm.at[psi_v.at[j]], ps_b, sem)

    def wait(j, tok_b, ps_b, sem):
      pltpu.make_async_copy(tok_hbm.at[idx_v.at[j]], tok_b, sem).wait()
      pltpu.make_async_copy(ps_hbm.at[psi_v.at[j]], ps_b, sem).wait()

    def process(j, tok_b, ps_b):
      def addr(r, c2):
        for c in range(EMBED // 16):
          sl = pl.ds(c * 16, 16)
          tok_b[r, sl] = tok_b[r, sl] + ps_b[r, sl]
        return c2
      lax.fori_loop(0, CHUNK, addr, 0)
      pltpu.sync_copy(tok_b, out_hbm.at[pl.ds(flat0 + j * CHUNK, CHUNK)])

    start(0, tok0, ps0, sem0)

    def pair(j2, carry):
      j = 2 * j2
      start(j + 1, tok1, ps1, sem1)
      wait(j, tok0, ps0, sem0)
      process(j, tok0, ps0)

      @pl.when(j2 < CHUNKS_PER_W // 2 - 1)
      def _():
        start(j + 2, tok0, ps0, sem0)
      wait(j + 1, tok1, ps1, sem1)
      process(j + 1, tok1, ps1)
      return carry
    lax.fori_loop(0, CHUNKS_PER_W // 2, pair, 0)

  return k(seq2d, lab2d, token_emb, ps_table)


def kernel(sequence, segment_labels, token_emb, position_emb, segment_emb):
  ps_table = (position_emb[:, None, :] + segment_emb[None, :, :]).reshape(
      SEQ * 3, EMBED)
  seq2d = sequence.reshape(N // CHUNK, CHUNK)
  lab2d = segment_labels.reshape(N // CHUNK, CHUNK)
  out = _sc_embed(seq2d, lab2d, token_emb, ps_table)
  return out.reshape(BATCH, SEQ, EMBED)


# speedup vs baseline: 2.3853x; 1.0253x over previous
import functools

import jax
import jax.numpy as jnp
from jax import lax
from jax.experimental import pallas as pl
from jax.experimental.pallas import tpu as pltpu
from jax.experimental.pallas import tpu_sc as plsc

VOCAB = 1000000
EMBED = 64
SEQ = 200
BATCH = 4096

NC, NS = 2, 16
NW = NC * NS
N = BATCH * SEQ
CHUNK = 64
ROWS_PER_W = N // NW
STAGE_ROWS = ROWS_PER_W // 128
CHUNKS_PER_W = ROWS_PER_W // CHUNK


def _sc_embed(seq2, lab2, tok2, ps2):
  mesh = plsc.VectorSubcoreMesh(core_axis_name="c", subcore_axis_name="s")

  @functools.partial(
      pl.kernel,
      mesh=mesh,
      compiler_params=pltpu.CompilerParams(use_tc_tiling_on_sc=True),
      out_type=jax.ShapeDtypeStruct((N, EMBED), jnp.float32),
      scratch_types=[
          pltpu.VMEM((STAGE_ROWS, 128), jnp.int32),
          pltpu.VMEM((STAGE_ROWS, 128), jnp.int32),
          pltpu.VMEM((300, 128), jnp.float32),
          pltpu.VMEM((CHUNK, 128), jnp.float32),
          pltpu.VMEM((CHUNK, 128), jnp.float32),
          pltpu.VMEM((CHUNK, EMBED), jnp.float32),
          pltpu.VMEM((CHUNK, EMBED), jnp.float32),
          pltpu.SemaphoreType.DMA,
          pltpu.SemaphoreType.DMA,
      ],
  )
  def k(seq_hbm, lab_hbm, tok_hbm, ps_hbm, out_hbm,
        idx_v, meta_v, ps_v, tok0, tok1, out0, out1, sem0, sem1):
    wid = lax.axis_index("s") * NC + lax.axis_index("c")
    r0 = wid * STAGE_ROWS
    flat0 = wid * ROWS_PER_W

    pltpu.sync_copy(seq_hbm.at[pl.ds(r0, STAGE_ROWS)], idx_v)
    pltpu.sync_copy(lab_hbm.at[pl.ds(r0, STAGE_ROWS)], meta_v)
    pltpu.sync_copy(ps_hbm, ps_v)

    iota = lax.iota(jnp.int32, 16)

    def fuse(r, carry):
      base = flat0 + r * 128
      for i in range(8):
        sl = pl.ds(i * 16, 16)
        pos = (base + i * 16 + iota) % SEQ
        v = idx_v[r, sl]
        meta_v[r, sl] = (pos * 3 + meta_v[r, sl]) * 2 + (v & 1)
        idx_v[r, sl] = v >> 1
      return carry
    lax.fori_loop(0, STAGE_ROWS, fuse, 0)

    def start(j, tok_b, sem):
      pltpu.async_copy(
          tok_hbm.at[idx_v.at[j // 2, pl.ds((j % 2) * CHUNK, CHUNK)]],
          tok_b, sem)

    def wait(j, tok_b, sem):
      pltpu.make_async_copy(
          tok_hbm.at[idx_v.at[j // 2, pl.ds((j % 2) * CHUNK, CHUNK)]],
          tok_b, sem).wait()

    def process(j, tok_b, out_b):
      def group(g, carry):
        mvec = meta_v[j // 2, pl.ds((j % 2) * CHUNK + g * 16, 16)]
        for jj in range(16):
          m = mvec[jj]
          h64 = (m & 1) * EMBED
          psf = (m >> 1) * EMBED
          row = g * 16 + jj
          for c in range(EMBED // 16):
            t16 = tok_b[row, pl.ds(h64 + c * 16, 16)]
            p16 = ps_v[(psf + c * 16) >> 7, pl.ds((psf + c * 16) & 127, 16)]
            out_b[row, pl.ds(c * 16, 16)] = t16 + p16
        return carry
      lax.fori_loop(0, CHUNK // 16, group, 0)
      pltpu.sync_copy(out_b, out_hbm.at[pl.ds(flat0 + j * CHUNK, CHUNK)])

    start(0, tok0, sem0)

    def pair(j2, carry):
      j = 2 * j2
      start(j + 1, tok1, sem1)
      wait(j, tok0, sem0)
      process(j, tok0, out0)

      @pl.when(j2 < CHUNKS_PER_W // 2 - 1)
      def _():
        start(j + 2, tok0, sem0)
      wait(j + 1, tok1, sem1)
      process(j + 1, tok1, out1)
      return carry
    lax.fori_loop(0, CHUNKS_PER_W // 2, pair, 0)

  return k(seq2, lab2, tok2, ps2)


def kernel(sequence, segment_labels, token_emb, position_emb, segment_emb):
  ps2 = (position_emb[:, None, :] + segment_emb[None, :, :]).reshape(300, 128)
  seq2 = sequence.reshape(N // 128, 128)
  lab2 = segment_labels.reshape(N // 128, 128)
  tok2 = token_emb.reshape(VOCAB // 2, 2 * EMBED)
  out = _sc_embed(seq2, lab2, tok2, ps2)
  return out.reshape(BATCH, SEQ, EMBED)


# speedup vs baseline: 2.4259x; 1.0170x over previous
import functools

import jax
import jax.numpy as jnp
from jax import lax
from jax.experimental import pallas as pl
from jax.experimental.pallas import tpu as pltpu
from jax.experimental.pallas import tpu_sc as plsc

VOCAB = 1000000
EMBED = 64
SEQ = 200
BATCH = 4096

NC, NS = 2, 16
NW = NC * NS
N = BATCH * SEQ
CHUNK = 128
ROWS_PER_W = N // NW
CHUNKS_PER_W = ROWS_PER_W // CHUNK


def _sc_embed(seq2, lab2, tok2, pos2, seg2):
  mesh = plsc.VectorSubcoreMesh(core_axis_name="c", subcore_axis_name="s")

  @functools.partial(
      pl.kernel,
      mesh=mesh,
      compiler_params=pltpu.CompilerParams(use_tc_tiling_on_sc=True),
      out_type=jax.ShapeDtypeStruct((N, EMBED), jnp.float32),
      scratch_types=[
          pltpu.VMEM((CHUNKS_PER_W, CHUNK), jnp.int32),
          pltpu.VMEM((CHUNKS_PER_W, CHUNK), jnp.int32),
          pltpu.VMEM((SEQ * EMBED // 128, 128), jnp.float32),
          pltpu.VMEM((2, 128), jnp.float32),
          pltpu.VMEM((CHUNK, 128), jnp.float32),
          pltpu.VMEM((CHUNK, 128), jnp.float32),
          pltpu.VMEM((CHUNK, EMBED), jnp.float32),
          pltpu.VMEM((CHUNK, EMBED), jnp.float32),
          pltpu.SemaphoreType.DMA,
          pltpu.SemaphoreType.DMA,
          pltpu.SemaphoreType.DMA,
          pltpu.SemaphoreType.DMA,
      ],
  )
  def k(seq_hbm, lab_hbm, tok_hbm, pos_hbm, seg_hbm, out_hbm,
        idx_v, meta_v, pos_v, seg_v, tok0, tok1, out0, out1,
        gs0, gs1, ss0, ss1):
    wid = lax.axis_index("s") * NC + lax.axis_index("c")
    r0 = pl.multiple_of(wid * CHUNKS_PER_W, 8)
    flat0 = pl.multiple_of(wid * ROWS_PER_W, 1024)

    pltpu.sync_copy(seq_hbm.at[pl.ds(r0, CHUNKS_PER_W)], idx_v)
    pltpu.sync_copy(lab_hbm.at[pl.ds(r0, CHUNKS_PER_W)], meta_v)
    pltpu.sync_copy(pos_hbm, pos_v)
    pltpu.sync_copy(seg_hbm, seg_v)

    iota = lax.iota(jnp.int32, 16)

    def fuse(r, carry):
      base = flat0 + r * CHUNK
      for i in range(CHUNK // 16):
        sl = pl.ds(i * 16, 16)
        pos = (base + i * 16 + iota) % SEQ
        v = idx_v[r, sl]
        meta_v[r, sl] = (pos * 4 + meta_v[r, sl]) * 2 + (v & 1)
        idx_v[r, sl] = v >> 1
      return carry
    lax.fori_loop(0, CHUNKS_PER_W, fuse, 0)

    def g_start(j, tok_b, sem):
      pltpu.async_copy(tok_hbm.at[idx_v.at[j]], tok_b, sem)

    def g_wait(j, tok_b, sem):
      pltpu.make_async_copy(tok_hbm.at[idx_v.at[j]], tok_b, sem).wait()

    def s_start(j, out_b, sem):
      off = pl.multiple_of(flat0 + j * CHUNK, 8)
      pltpu.async_copy(out_b, out_hbm.at[pl.ds(off, CHUNK)], sem)

    def s_wait(j, out_b, sem):
      off = pl.multiple_of(flat0 + j * CHUNK, 8)
      pltpu.make_async_copy(out_b, out_hbm.at[pl.ds(off, CHUNK)], sem).wait()

    def process(j, tok_b, out_b):
      def group(g, carry):
        mvec = meta_v[j, pl.ds(g * 16, 16)]
        for jj in range(16):
          m = mvec[jj]
          h64 = (m & 1) * EMBED
          mm = m >> 1
          posf = (mm >> 2) * EMBED
          segf = (mm & 3) * EMBED
          row = g * 16 + jj
          for c in range(EMBED // 16):
            t16 = tok_b[row, pl.ds(h64 + c * 16, 16)]
            pf = posf + c * 16
            p16 = pos_v[pf >> 7, pl.ds(pf & 127, 16)]
            sf = segf + c * 16
            s16 = seg_v[sf >> 7, pl.ds(sf & 127, 16)]
            out_b[row, pl.ds(c * 16, 16)] = t16 + p16 + s16
        return carry
      lax.fori_loop(0, CHUNK // 16, group, 0)

    g_start(0, tok0, gs0)

    def pair(i, carry):
      j = 2 * i
      g_start(j + 1, tok1, gs1)

      @pl.when(i > 0)
      def _():
        s_wait(j - 2, out0, ss0)
      g_wait(j, tok0, gs0)
      process(j, tok0, out0)

      @pl.when(i < CHUNKS_PER_W // 2 - 1)
      def _():
        g_start(j + 2, tok0, gs0)
      s_start(j, out0, ss0)

      @pl.when(i > 0)
      def _():
        s_wait(j - 1, out1, ss1)
      g_wait(j + 1, tok1, gs1)
      process(j + 1, tok1, out1)
      s_start(j + 1, out1, ss1)
      return carry
    lax.fori_loop(0, CHUNKS_PER_W // 2, pair, 0)
    s_wait(CHUNKS_PER_W - 2, out0, ss0)
    s_wait(CHUNKS_PER_W - 1, out1, ss1)

  return k(seq2, lab2, tok2, pos2, seg2)


def kernel(sequence, segment_labels, token_emb, position_emb, segment_emb):
  seq2 = sequence.reshape(N // CHUNK, CHUNK)
  lab2 = segment_labels.reshape(N // CHUNK, CHUNK)
  tok2 = token_emb.reshape(VOCAB // 2, 2 * EMBED)
  pos2 = position_emb.reshape(SEQ * EMBED // 128, 128)
  seg2 = jnp.pad(segment_emb.reshape(3 * EMBED), (0, 64)).reshape(2, 128)
  out = _sc_embed(seq2, lab2, tok2, pos2, seg2)
  return out.reshape(BATCH, SEQ, EMBED)
